# baseline (speedup 1.0000x reference)
import jax
import jax.numpy as jnp
from jax import lax
from jax.experimental import pallas as pl

_ROWS = 128
_N = 32768
_BR = 8

_INT_MIN = -(2**31)
_INT_MAX = 2**31 - 1


def _body(klen_ref, p_ref, g_ref, out_ref):
    conf = jnp.log(p_ref[...]) + g_ref[...]
    b = lax.bitcast_convert_type(conf, jnp.int32)
    key = b ^ (lax.shift_right_arithmetic(b, 31) & jnp.int32(0x7FFFFFFF))

    kp1 = klen_ref[...] + jnp.int32(1)
    lo0 = jnp.full((_BR, 1), _INT_MIN, jnp.int32)
    hi0 = jnp.full((_BR, 1), _INT_MAX, jnp.int32)
    ones = jnp.ones((_N, 1), jnp.float32)

    def step(_, carry):
        lo, hi = carry
        mid = (lo & hi) + lax.shift_right_arithmetic(lo ^ hi, 1)
        x = (key <= mid).astype(jnp.float32)
        cnt = jax.lax.dot_general(
            x, ones, (((1,), (0,)), ((), ())),
            preferred_element_type=jnp.float32).astype(jnp.int32)
        ge = cnt >= kp1
        return (jnp.where(ge, lo, mid + 1), jnp.where(ge, mid, hi))

    lo, _ = lax.fori_loop(0, 32, step, (lo0, hi0))
    out_ref[...] = key < lo


def kernel(mask_len, probs):
    gumbel = jax.random.gumbel(jax.random.key(42), probs.shape, probs.dtype)
    return pl.pallas_call(
        _body,
        grid=(_ROWS // _BR,),
        in_specs=[
            pl.BlockSpec((_BR, 1), lambda i: (i, 0)),
            pl.BlockSpec((_BR, _N), lambda i: (i, 0)),
            pl.BlockSpec((_BR, _N), lambda i: (i, 0)),
        ],
        out_specs=pl.BlockSpec((_BR, _N), lambda i: (i, 0)),
        out_shape=jax.ShapeDtypeStruct((_ROWS, _N), jnp.bool_),
    )(mask_len, probs, gumbel)


# speedup vs baseline: 2.4556x; 2.4556x over previous
import jax
import jax.numpy as jnp
from jax import lax
from jax.experimental import pallas as pl

_ROWS = 128
_N = 32768
_BR = 8

_INT_MIN = -(2**31)
_INT_MAX = 2**31 - 1


def _body(klen_ref, p_ref, g_ref, out_ref):
    conf = jnp.log(p_ref[...]) + g_ref[...]
    b = lax.bitcast_convert_type(conf, jnp.int32)
    key = b ^ (lax.shift_right_arithmetic(b, 31) & jnp.int32(0x7FFFFFFF))

    kp1 = klen_ref[...] + jnp.int32(1)
    lo = jnp.full((_BR, 1), _INT_MIN, jnp.int32)
    hi = jnp.full((_BR, 1), _INT_MAX, jnp.int32)

    for _ in range(32):
        mid = (lo & hi) + lax.shift_right_arithmetic(lo ^ hi, 1)
        cnt = jnp.sum((key <= mid).astype(jnp.int32), axis=1, keepdims=True)
        ge = cnt >= kp1
        lo, hi = jnp.where(ge, lo, mid + 1), jnp.where(ge, mid, hi)
    out_ref[...] = key < lo


def kernel(mask_len, probs):
    gumbel = jax.random.gumbel(jax.random.key(42), probs.shape, probs.dtype)
    return pl.pallas_call(
        _body,
        grid=(_ROWS // _BR,),
        in_specs=[
            pl.BlockSpec((_BR, 1), lambda i: (i, 0)),
            pl.BlockSpec((_BR, _N), lambda i: (i, 0)),
            pl.BlockSpec((_BR, _N), lambda i: (i, 0)),
        ],
        out_specs=pl.BlockSpec((_BR, _N), lambda i: (i, 0)),
        out_shape=jax.ShapeDtypeStruct((_ROWS, _N), jnp.bool_),
    )(mask_len, probs, gumbel)


# speedup vs baseline: 3.4462x; 1.4034x over previous
import jax
import jax.numpy as jnp
from jax import lax
from jax.experimental import pallas as pl

_ROWS = 128
_N = 32768
_BR = 16

_INT_MIN = -(2**31)
_INT_MAX = 2**31 - 1


def _body(klen_ref, p_ref, g_ref, out_ref):
    conf = jnp.log(p_ref[...]) + g_ref[...]
    b = lax.bitcast_convert_type(conf, jnp.int32)
    key = b ^ (lax.shift_right_arithmetic(b, 31) & jnp.int32(0x7FFFFFFF))

    kp1 = klen_ref[...] + jnp.int32(1)
    lo = jnp.full((_BR, 1), _INT_MIN, jnp.int32)
    hi = jnp.full((_BR, 1), _INT_MAX, jnp.int32)

    for _ in range(32):
        mid = (lo & hi) + lax.shift_right_arithmetic(lo ^ hi, 1)
        cnt = jnp.sum((key <= mid).astype(jnp.int32), axis=1, keepdims=True)
        ge = cnt >= kp1
        lo, hi = jnp.where(ge, lo, mid + 1), jnp.where(ge, mid, hi)
    out_ref[...] = key < lo


def kernel(mask_len, probs):
    gumbel = jax.random.gumbel(jax.random.key(42), probs.shape, probs.dtype)
    return pl.pallas_call(
        _body,
        grid=(_ROWS // _BR,),
        in_specs=[
            pl.BlockSpec((_BR, 1), lambda i: (i, 0)),
            pl.BlockSpec((_BR, _N), lambda i: (i, 0)),
            pl.BlockSpec((_BR, _N), lambda i: (i, 0)),
        ],
        out_specs=pl.BlockSpec((_BR, _N), lambda i: (i, 0)),
        out_shape=jax.ShapeDtypeStruct((_ROWS, _N), jnp.bool_),
    )(mask_len, probs, gumbel)


# speedup vs baseline: 4.0488x; 1.1749x over previous
import jax
import jax.numpy as jnp
from jax import lax
from jax.experimental import pallas as pl

_ROWS = 128
_N = 32768
_BR = 32

_INT_MIN = -(2**31)
_INT_MAX = 2**31 - 1


def _body(klen_ref, p_ref, g_ref, out_ref):
    conf = jnp.log(p_ref[...]) + g_ref[...]
    b = lax.bitcast_convert_type(conf, jnp.int32)
    key = b ^ (lax.shift_right_arithmetic(b, 31) & jnp.int32(0x7FFFFFFF))

    kp1 = klen_ref[...] + jnp.int32(1)
    lo = jnp.full((_BR, 1), _INT_MIN, jnp.int32)
    hi = jnp.full((_BR, 1), _INT_MAX, jnp.int32)

    for _ in range(32):
        mid = (lo & hi) + lax.shift_right_arithmetic(lo ^ hi, 1)
        cnt = jnp.sum((key <= mid).astype(jnp.int32), axis=1, keepdims=True)
        ge = cnt >= kp1
        lo, hi = jnp.where(ge, lo, mid + 1), jnp.where(ge, mid, hi)
    out_ref[...] = key < lo


def kernel(mask_len, probs):
    gumbel = jax.random.gumbel(jax.random.key(42), probs.shape, probs.dtype)
    return pl.pallas_call(
        _body,
        grid=(_ROWS // _BR,),
        in_specs=[
            pl.BlockSpec((_BR, 1), lambda i: (i, 0)),
            pl.BlockSpec((_BR, _N), lambda i: (i, 0)),
            pl.BlockSpec((_BR, _N), lambda i: (i, 0)),
        ],
        out_specs=pl.BlockSpec((_BR, _N), lambda i: (i, 0)),
        out_shape=jax.ShapeDtypeStruct((_ROWS, _N), jnp.bool_),
    )(mask_len, probs, gumbel)
